# baseline (speedup 1.0000x reference)
import functools

import jax
import jax.numpy as jnp
from jax import lax
from jax.experimental import pallas as pl
from jax.experimental.pallas import tpu as pltpu
from jax.experimental.pallas import tpu_sc as plsc

_B = 4096
_D = 128
_L = 16
_NC = 2
_NS = 16
_NW = _NC * _NS
_BPW = _B // _NW
_NCH = _D // _L


_NCK = 2
_CR = _BPW // _NCK
_U = 4


def _tec_body(h_hbm, r_hbm, t_hbm, ent_hbm, rel_hbm, nrm_hbm, out_hbm,
              hidx, ridx, tidx, hrows, trows, rrows, wrows, outv, oscr,
              sem_idx, *sems):
    wid = lax.axis_index("s") * _NC + lax.axis_index("c")
    base = wid * _BPW

    ci_h = pltpu.async_copy(h_hbm.at[pl.ds(base, _BPW)], hidx, sem_idx)
    ci_t = pltpu.async_copy(t_hbm.at[pl.ds(base, _BPW)], tidx, sem_idx)
    ci_r = pltpu.async_copy(r_hbm.at[pl.ds(base, _BPW)], ridx, sem_idx)
    ci_h.wait()
    ci_t.wait()
    ci_r.wait()

    copies = []
    for c in range(_NCK):
        sl = pl.ds(c * _CR, _CR)
        copies.append((
            pltpu.async_copy(ent_hbm.at[hidx.at[sl]], hrows.at[sl], sems[c]),
            pltpu.async_copy(ent_hbm.at[tidx.at[sl]], trows.at[sl], sems[c]),
            pltpu.async_copy(rel_hbm.at[ridx.at[sl]], rrows.at[sl], sems[c]),
            pltpu.async_copy(nrm_hbm.at[ridx.at[sl]], wrows.at[sl], sems[c]),
        ))

    lane = lax.broadcasted_iota(jnp.int32, (_L,), 0)
    _dnums = lax.GatherDimensionNumbers(
        offset_dims=(), collapsed_slice_dims=(0,), start_index_map=(0,))

    def shuffle(x, idx):
        return lax.gather(x, idx[:, None], _dnums, slice_sizes=(1,),
                          mode=lax.GatherScatterMode.PROMISE_IN_BOUNDS)

    def allsum(x):
        for d in (8, 4, 2, 1):
            x = x + shuffle(x, lane ^ d)
        return x

    def quad(g):
        svec = jnp.zeros((_L,), jnp.float32)
        for u in range(_U):
            i = g * _U + u
            dw0 = jnp.zeros((_L,), jnp.float32)
            dw1 = jnp.zeros((_L,), jnp.float32)
            ww0 = jnp.zeros((_L,), jnp.float32)
            ww1 = jnp.zeros((_L,), jnp.float32)
            dch = []
            wch = []
            for k in range(_NCH):
                hk = hrows[i, pl.ds(k * _L, _L)]
                tk = trows[i, pl.ds(k * _L, _L)]
                wk = wrows[i, pl.ds(k * _L, _L)]
                dk = hk - tk
                if k % 2 == 0:
                    dw0 = dw0 + dk * wk
                    ww0 = ww0 + wk * wk
                else:
                    dw1 = dw1 + dk * wk
                    ww1 = ww1 + wk * wk
                dch.append(dk)
                wch.append(wk)
            c = allsum(dw0 + dw1) / jnp.maximum(allsum(ww0 + ww1), 1e-24)
            acc0 = jnp.zeros((_L,), jnp.float32)
            acc1 = jnp.zeros((_L,), jnp.float32)
            for k in range(_NCH):
                rk = rrows[i, pl.ds(k * _L, _L)]
                term = jnp.abs(dch[k] + rk - c * wch[k])
                if k % 2 == 0:
                    acc0 = acc0 + term
                else:
                    acc1 = acc1 + term
            svec = jnp.where(lane == u, -allsum(acc0 + acc1), svec)
        oscr[g, :] = svec

    for c in range(_NCK):
        for cp in copies[c]:
            cp.wait()
        g0 = c * (_CR // _U)
        plsc.parallel_loop(g0, g0 + _CR // _U)(quad)

    per = _L // _U
    for j in range(_BPW // _L):
        merged = jnp.zeros((_L,), jnp.float32)
        for m in range(per):
            blk = oscr[j * per + m, :]
            rot = shuffle(blk, (lane - m * _U) & (_L - 1))
            merged = jnp.where((lane >= m * _U) & (lane < (m + 1) * _U),
                               rot, merged)
        outv[pl.ds(j * _L, _L)] = merged
    pltpu.sync_copy(outv, out_hbm.at[pl.ds(base, _BPW)])


@jax.jit
def _transh_sc(h, r, t, ent_emb, rel_emb, rel_norm):
    mesh = plsc.VectorSubcoreMesh(core_axis_name="c", subcore_axis_name="s")
    run = functools.partial(
        pl.kernel,
        mesh=mesh,
        out_type=jax.ShapeDtypeStruct((_B,), jnp.float32),
        scratch_types=[
            pltpu.VMEM((_BPW,), jnp.int32),
            pltpu.VMEM((_BPW,), jnp.int32),
            pltpu.VMEM((_BPW,), jnp.int32),
            pltpu.VMEM((_BPW, _D), jnp.float32),
            pltpu.VMEM((_BPW, _D), jnp.float32),
            pltpu.VMEM((_BPW, _D), jnp.float32),
            pltpu.VMEM((_BPW, _D), jnp.float32),
            pltpu.VMEM((_BPW,), jnp.float32),
            pltpu.VMEM((_BPW // _U, _L), jnp.float32),
            pltpu.SemaphoreType.DMA,
            pltpu.SemaphoreType.DMA,
            pltpu.SemaphoreType.DMA,
        ],
    )(_tec_body)
    return run(h, r, t, ent_emb, rel_emb, rel_norm)


def kernel(h, r, t, ent_emb, rel_emb, rel_norm):
    score = _transh_sc(h.astype(jnp.int32), r.astype(jnp.int32),
                       t.astype(jnp.int32), ent_emb, rel_emb, rel_norm)
    return score.reshape(_B, 1)


# speedup vs baseline: 1.3213x; 1.3213x over previous
import functools

import jax
import jax.numpy as jnp
from jax import lax
from jax.experimental import pallas as pl
from jax.experimental.pallas import tpu as pltpu
from jax.experimental.pallas import tpu_sc as plsc

_B = 4096
_D = 128
_L = 16
_NC = 2
_NS = 16
_NW = _NC * _NS
_BPW = _B // _NW
_NCH = _D // _L


_NCK = 4
_CR = _BPW // _NCK
_U = 4


def _tec_body(h_hbm, r_hbm, t_hbm, ent_hbm, rel_hbm, nrm_hbm, out_hbm,
              hidx, ridx, tidx, hrows, trows, rrows, wrows, outv,
              sem_idx, *sems):
    wid = lax.axis_index("s") * _NC + lax.axis_index("c")
    base = wid * _BPW

    ci_h = pltpu.async_copy(h_hbm.at[pl.ds(base, _BPW)], hidx, sem_idx)
    ci_t = pltpu.async_copy(t_hbm.at[pl.ds(base, _BPW)], tidx, sem_idx)
    ci_r = pltpu.async_copy(r_hbm.at[pl.ds(base, _BPW)], ridx, sem_idx)
    ci_h.wait()
    ci_t.wait()
    ci_r.wait()

    copies = []
    for c in range(_NCK):
        sl = pl.ds(c * _CR, _CR)
        copies.append((
            pltpu.async_copy(ent_hbm.at[hidx.at[sl]], hrows.at[sl], sems[c]),
            pltpu.async_copy(ent_hbm.at[tidx.at[sl]], trows.at[sl], sems[c]),
            pltpu.async_copy(rel_hbm.at[ridx.at[sl]], rrows.at[sl], sems[c]),
            pltpu.async_copy(nrm_hbm.at[ridx.at[sl]], wrows.at[sl], sems[c]),
        ))

    lane = lax.broadcasted_iota(jnp.int32, (_L,), 0)
    _dnums = lax.GatherDimensionNumbers(
        offset_dims=(), collapsed_slice_dims=(0,), start_index_map=(0,))

    def shuffle(x, idx):
        return lax.gather(x, idx[:, None], _dnums, slice_sizes=(1,),
                          mode=lax.GatherScatterMode.PROMISE_IN_BOUNDS)

    def allsum(x):
        for d in (8, 4, 2, 1):
            x = x + shuffle(x, lane ^ d)
        return x

    def quad(g, svec):
        for u in range(_U):
            i = g * _U + u
            dw0 = jnp.zeros((_L,), jnp.float32)
            dw1 = jnp.zeros((_L,), jnp.float32)
            ww0 = jnp.zeros((_L,), jnp.float32)
            ww1 = jnp.zeros((_L,), jnp.float32)
            dch = []
            wch = []
            for k in range(_NCH):
                hk = hrows[i, pl.ds(k * _L, _L)]
                tk = trows[i, pl.ds(k * _L, _L)]
                wk = wrows[i, pl.ds(k * _L, _L)]
                dk = hk - tk
                if k % 2 == 0:
                    dw0 = dw0 + dk * wk
                    ww0 = ww0 + wk * wk
                else:
                    dw1 = dw1 + dk * wk
                    ww1 = ww1 + wk * wk
                dch.append(dk)
                wch.append(wk)
            c = allsum(dw0 + dw1) / jnp.maximum(allsum(ww0 + ww1), 1e-24)
            acc0 = jnp.zeros((_L,), jnp.float32)
            acc1 = jnp.zeros((_L,), jnp.float32)
            for k in range(_NCH):
                rk = rrows[i, pl.ds(k * _L, _L)]
                term = jnp.abs(dch[k] + rk - c * wch[k])
                if k % 2 == 0:
                    acc0 = acc0 + term
                else:
                    acc1 = acc1 + term
            lpos = (g % (_L // _U)) * _U + u
            svec = jnp.where(lane == lpos, -allsum(acc0 + acc1), svec)
        outv[pl.ds((g // (_L // _U)) * _L, _L)] = svec
        return svec

    gpc = _CR // _U
    svec = jnp.zeros((_L,), jnp.float32)

    def step(g, svec):
        for c in range(1, _NCK):
            @pl.when(g == c * gpc)
            def _():
                for cp in copies[c]:
                    cp.wait()
        return quad(g, svec)

    for cp in copies[0]:
        cp.wait()
    lax.fori_loop(0, _NCK * gpc, step, svec)
    pltpu.sync_copy(outv, out_hbm.at[pl.ds(base, _BPW)])


@jax.jit
def _transh_sc(h, r, t, ent_emb, rel_emb, rel_norm):
    mesh = plsc.VectorSubcoreMesh(core_axis_name="c", subcore_axis_name="s")
    run = functools.partial(
        pl.kernel,
        mesh=mesh,
        out_type=jax.ShapeDtypeStruct((_B,), jnp.float32),
        scratch_types=[
            pltpu.VMEM((_BPW,), jnp.int32),
            pltpu.VMEM((_BPW,), jnp.int32),
            pltpu.VMEM((_BPW,), jnp.int32),
            pltpu.VMEM((_BPW, _D), jnp.float32),
            pltpu.VMEM((_BPW, _D), jnp.float32),
            pltpu.VMEM((_BPW, _D), jnp.float32),
            pltpu.VMEM((_BPW, _D), jnp.float32),
            pltpu.VMEM((_BPW,), jnp.float32),
            pltpu.SemaphoreType.DMA,
            pltpu.SemaphoreType.DMA,
            pltpu.SemaphoreType.DMA,
            pltpu.SemaphoreType.DMA,
            pltpu.SemaphoreType.DMA,
        ],
    )(_tec_body)
    return run(h, r, t, ent_emb, rel_emb, rel_norm)


def kernel(h, r, t, ent_emb, rel_emb, rel_norm):
    score = _transh_sc(h.astype(jnp.int32), r.astype(jnp.int32),
                       t.astype(jnp.int32), ent_emb, rel_emb, rel_norm)
    return score.reshape(_B, 1)


# speedup vs baseline: 1.3231x; 1.0013x over previous
import functools

import jax
import jax.numpy as jnp
from jax import lax
from jax.experimental import pallas as pl
from jax.experimental.pallas import tpu as pltpu
from jax.experimental.pallas import tpu_sc as plsc

_B = 4096
_D = 128
_L = 16
_NC = 2
_NS = 16
_NW = _NC * _NS
_BPW = _B // _NW
_NCH = _D // _L


_NCK = 4
_CR = _BPW // _NCK
_U = 8


def _tec_body(h_hbm, r_hbm, t_hbm, ent_hbm, rel_hbm, nrm_hbm, out_hbm,
              hidx, ridx, tidx, hrows, trows, rrows, wrows, outv,
              sem_idx, *sems):
    wid = lax.axis_index("s") * _NC + lax.axis_index("c")
    base = wid * _BPW

    ci_h = pltpu.async_copy(h_hbm.at[pl.ds(base, _BPW)], hidx, sem_idx)
    ci_t = pltpu.async_copy(t_hbm.at[pl.ds(base, _BPW)], tidx, sem_idx)
    ci_r = pltpu.async_copy(r_hbm.at[pl.ds(base, _BPW)], ridx, sem_idx)
    ci_h.wait()
    ci_t.wait()
    ci_r.wait()

    copies = []
    for c in range(_NCK):
        sl = pl.ds(c * _CR, _CR)
        copies.append((
            pltpu.async_copy(ent_hbm.at[hidx.at[sl]], hrows.at[sl], sems[c]),
            pltpu.async_copy(ent_hbm.at[tidx.at[sl]], trows.at[sl], sems[c]),
            pltpu.async_copy(rel_hbm.at[ridx.at[sl]], rrows.at[sl], sems[c]),
            pltpu.async_copy(nrm_hbm.at[ridx.at[sl]], wrows.at[sl], sems[c]),
        ))

    lane = lax.broadcasted_iota(jnp.int32, (_L,), 0)
    _dnums = lax.GatherDimensionNumbers(
        offset_dims=(), collapsed_slice_dims=(0,), start_index_map=(0,))

    def shuffle(x, idx):
        return lax.gather(x, idx[:, None], _dnums, slice_sizes=(1,),
                          mode=lax.GatherScatterMode.PROMISE_IN_BOUNDS)

    def allsum(x):
        for d in (8, 4, 2, 1):
            x = x + shuffle(x, lane ^ d)
        return x

    def quad(g, svec):
        for u in range(_U):
            i = g * _U + u
            dw0 = jnp.zeros((_L,), jnp.float32)
            dw1 = jnp.zeros((_L,), jnp.float32)
            ww0 = jnp.zeros((_L,), jnp.float32)
            ww1 = jnp.zeros((_L,), jnp.float32)
            dch = []
            wch = []
            for k in range(_NCH):
                hk = hrows[i, pl.ds(k * _L, _L)]
                tk = trows[i, pl.ds(k * _L, _L)]
                wk = wrows[i, pl.ds(k * _L, _L)]
                dk = hk - tk
                if k % 2 == 0:
                    dw0 = dw0 + dk * wk
                    ww0 = ww0 + wk * wk
                else:
                    dw1 = dw1 + dk * wk
                    ww1 = ww1 + wk * wk
                dch.append(dk)
                wch.append(wk)
            c = allsum(dw0 + dw1) / jnp.maximum(allsum(ww0 + ww1), 1e-24)
            acc0 = jnp.zeros((_L,), jnp.float32)
            acc1 = jnp.zeros((_L,), jnp.float32)
            for k in range(_NCH):
                rk = rrows[i, pl.ds(k * _L, _L)]
                term = jnp.abs(dch[k] + rk - c * wch[k])
                if k % 2 == 0:
                    acc0 = acc0 + term
                else:
                    acc1 = acc1 + term
            lpos = (g % (_L // _U)) * _U + u
            svec = jnp.where(lane == lpos, -allsum(acc0 + acc1), svec)
        outv[pl.ds((g // (_L // _U)) * _L, _L)] = svec
        return svec

    gpc = _CR // _U
    svec = jnp.zeros((_L,), jnp.float32)

    def step(g, svec):
        for c in range(1, _NCK):
            @pl.when(g == c * gpc)
            def _():
                for cp in copies[c]:
                    cp.wait()
        return quad(g, svec)

    for cp in copies[0]:
        cp.wait()
    lax.fori_loop(0, _NCK * gpc, step, svec)
    pltpu.sync_copy(outv, out_hbm.at[pl.ds(base, _BPW)])


@jax.jit
def _transh_sc(h, r, t, ent_emb, rel_emb, rel_norm):
    mesh = plsc.VectorSubcoreMesh(core_axis_name="c", subcore_axis_name="s")
    run = functools.partial(
        pl.kernel,
        mesh=mesh,
        out_type=jax.ShapeDtypeStruct((_B,), jnp.float32),
        scratch_types=[
            pltpu.VMEM((_BPW,), jnp.int32),
            pltpu.VMEM((_BPW,), jnp.int32),
            pltpu.VMEM((_BPW,), jnp.int32),
            pltpu.VMEM((_BPW, _D), jnp.float32),
            pltpu.VMEM((_BPW, _D), jnp.float32),
            pltpu.VMEM((_BPW, _D), jnp.float32),
            pltpu.VMEM((_BPW, _D), jnp.float32),
            pltpu.VMEM((_BPW,), jnp.float32),
            pltpu.SemaphoreType.DMA,
            pltpu.SemaphoreType.DMA,
            pltpu.SemaphoreType.DMA,
            pltpu.SemaphoreType.DMA,
            pltpu.SemaphoreType.DMA,
        ],
    )(_tec_body)
    return run(h, r, t, ent_emb, rel_emb, rel_norm)


def kernel(h, r, t, ent_emb, rel_emb, rel_norm):
    score = _transh_sc(h.astype(jnp.int32), r.astype(jnp.int32),
                       t.astype(jnp.int32), ent_emb, rel_emb, rel_norm)
    return score.reshape(_B, 1)


# speedup vs baseline: 1.3523x; 1.0221x over previous
import functools

import jax
import jax.numpy as jnp
from jax import lax
from jax.experimental import pallas as pl
from jax.experimental.pallas import tpu as pltpu
from jax.experimental.pallas import tpu_sc as plsc

_B = 4096
_D = 128
_L = 16
_NC = 2
_NS = 16
_NW = _NC * _NS
_BPW = _B // _NW
_NCH = _D // _L


_NCK = 4
_CR = _BPW // _NCK
_U = 4


def _tec_body(h_hbm, r_hbm, t_hbm, ent_hbm, rel_hbm, nrm_hbm, out_hbm,
              hidx, ridx, tidx, hrows, trows, rrows, wrows, outv,
              sem_idx, *sems):
    wid = lax.axis_index("s") * _NC + lax.axis_index("c")
    base = wid * _BPW

    ci_h = pltpu.async_copy(h_hbm.at[pl.ds(base, _BPW)], hidx, sem_idx)
    ci_t = pltpu.async_copy(t_hbm.at[pl.ds(base, _BPW)], tidx, sem_idx)
    ci_r = pltpu.async_copy(r_hbm.at[pl.ds(base, _BPW)], ridx, sem_idx)
    ci_h.wait()
    ci_t.wait()
    ci_r.wait()

    copies = []
    for c in range(_NCK):
        sl = pl.ds(c * _CR, _CR)
        copies.append((
            pltpu.make_async_copy(ent_hbm.at[hidx.at[sl]], hrows.at[sl], sems[c]),
            pltpu.make_async_copy(ent_hbm.at[tidx.at[sl]], trows.at[sl], sems[c]),
            pltpu.make_async_copy(rel_hbm.at[ridx.at[sl]], rrows.at[sl], sems[c]),
            pltpu.make_async_copy(nrm_hbm.at[ridx.at[sl]], wrows.at[sl], sems[c]),
        ))
    for cp in copies[0]:
        cp.start()

    lane = lax.broadcasted_iota(jnp.int32, (_L,), 0)
    _dnums = lax.GatherDimensionNumbers(
        offset_dims=(), collapsed_slice_dims=(0,), start_index_map=(0,))

    def shuffle(x, idx):
        return lax.gather(x, idx[:, None], _dnums, slice_sizes=(1,),
                          mode=lax.GatherScatterMode.PROMISE_IN_BOUNDS)

    def allsum(x):
        for d in (8, 4, 2, 1):
            x = x + shuffle(x, lane ^ d)
        return x

    def quad(g, svec):
        for u in range(_U):
            i = g * _U + u
            dw0 = jnp.zeros((_L,), jnp.float32)
            dw1 = jnp.zeros((_L,), jnp.float32)
            ww0 = jnp.zeros((_L,), jnp.float32)
            ww1 = jnp.zeros((_L,), jnp.float32)
            dch = []
            wch = []
            for k in range(_NCH):
                hk = hrows[i, pl.ds(k * _L, _L)]
                tk = trows[i, pl.ds(k * _L, _L)]
                wk = wrows[i, pl.ds(k * _L, _L)]
                dk = hk - tk
                if k % 2 == 0:
                    dw0 = dw0 + dk * wk
                    ww0 = ww0 + wk * wk
                else:
                    dw1 = dw1 + dk * wk
                    ww1 = ww1 + wk * wk
                dch.append(dk)
                wch.append(wk)
            c = allsum(dw0 + dw1) / jnp.maximum(allsum(ww0 + ww1), 1e-24)
            acc0 = jnp.zeros((_L,), jnp.float32)
            acc1 = jnp.zeros((_L,), jnp.float32)
            for k in range(_NCH):
                rk = rrows[i, pl.ds(k * _L, _L)]
                term = jnp.abs(dch[k] + rk - c * wch[k])
                if k % 2 == 0:
                    acc0 = acc0 + term
                else:
                    acc1 = acc1 + term
            lpos = (g % (_L // _U)) * _U + u
            svec = jnp.where(lane == lpos, -allsum(acc0 + acc1), svec)
        outv[pl.ds((g // (_L // _U)) * _L, _L)] = svec
        return svec

    gpc = _CR // _U
    svec = jnp.zeros((_L,), jnp.float32)

    def step(g, svec):
        for c in range(_NCK):
            @pl.when(g == c * gpc)
            def _(c=c):
                for cp in copies[c]:
                    cp.wait()
                if c + 1 < _NCK:
                    for cp in copies[c + 1]:
                        cp.start()
        return quad(g, svec)

    lax.fori_loop(0, _NCK * gpc, step, svec)
    pltpu.sync_copy(outv, out_hbm.at[pl.ds(base, _BPW)])


@jax.jit
def _transh_sc(h, r, t, ent_emb, rel_emb, rel_norm):
    mesh = plsc.VectorSubcoreMesh(core_axis_name="c", subcore_axis_name="s")
    run = functools.partial(
        pl.kernel,
        mesh=mesh,
        out_type=jax.ShapeDtypeStruct((_B,), jnp.float32),
        scratch_types=[
            pltpu.VMEM((_BPW,), jnp.int32),
            pltpu.VMEM((_BPW,), jnp.int32),
            pltpu.VMEM((_BPW,), jnp.int32),
            pltpu.VMEM((_BPW, _D), jnp.float32),
            pltpu.VMEM((_BPW, _D), jnp.float32),
            pltpu.VMEM((_BPW, _D), jnp.float32),
            pltpu.VMEM((_BPW, _D), jnp.float32),
            pltpu.VMEM((_BPW,), jnp.float32),
            pltpu.SemaphoreType.DMA,
            pltpu.SemaphoreType.DMA,
            pltpu.SemaphoreType.DMA,
            pltpu.SemaphoreType.DMA,
            pltpu.SemaphoreType.DMA,
        ],
    )(_tec_body)
    return run(h, r, t, ent_emb, rel_emb, rel_norm)


def kernel(h, r, t, ent_emb, rel_emb, rel_norm):
    score = _transh_sc(h.astype(jnp.int32), r.astype(jnp.int32),
                       t.astype(jnp.int32), ent_emb, rel_emb, rel_norm)
    return score.reshape(_B, 1)


# speedup vs baseline: 1.3852x; 1.0243x over previous
import functools

import jax
import jax.numpy as jnp
from jax import lax
from jax.experimental import pallas as pl
from jax.experimental.pallas import tpu as pltpu
from jax.experimental.pallas import tpu_sc as plsc

_B = 4096
_D = 128
_L = 16
_NC = 2
_NS = 16
_NW = _NC * _NS
_BPW = _B // _NW
_NCH = _D // _L


_NCK = 4
_CR = _BPW // _NCK
_U = 4


def _tec_body(h_hbm, r_hbm, t_hbm, ent_hbm, rel_hbm, nrm_hbm, out_hbm,
              hidx, ridx, tidx, hrows, trows, rrows, wrows, outv,
              sem_idx, *sems):
    wid = lax.axis_index("s") * _NC + lax.axis_index("c")
    base = wid * _BPW

    ci_h = pltpu.async_copy(h_hbm.at[pl.ds(base, _BPW)], hidx, sem_idx)
    ci_t = pltpu.async_copy(t_hbm.at[pl.ds(base, _BPW)], tidx, sem_idx)
    ci_r = pltpu.async_copy(r_hbm.at[pl.ds(base, _BPW)], ridx, sem_idx)
    ci_h.wait()
    ci_t.wait()
    ci_r.wait()

    copies = []
    for c in range(_NCK):
        sl = pl.ds(c * _CR, _CR)
        copies.append((
            pltpu.make_async_copy(ent_hbm.at[hidx.at[sl]], hrows.at[sl], sems[c]),
            pltpu.make_async_copy(ent_hbm.at[tidx.at[sl]], trows.at[sl], sems[c]),
            pltpu.make_async_copy(rel_hbm.at[ridx.at[sl]], rrows.at[sl], sems[c]),
            pltpu.make_async_copy(nrm_hbm.at[ridx.at[sl]], wrows.at[sl], sems[c]),
        ))
    for cp in copies[0]:
        cp.start()
    for cp in copies[1]:
        cp.start()

    lane = lax.broadcasted_iota(jnp.int32, (_L,), 0)
    _dnums = lax.GatherDimensionNumbers(
        offset_dims=(), collapsed_slice_dims=(0,), start_index_map=(0,))

    def shuffle(x, idx):
        return lax.gather(x, idx[:, None], _dnums, slice_sizes=(1,),
                          mode=lax.GatherScatterMode.PROMISE_IN_BOUNDS)

    def allsum(x):
        for d in (8, 4, 2, 1):
            x = x + shuffle(x, lane ^ d)
        return x

    def quad(g, svec):
        for u in range(_U):
            i = g * _U + u
            dw0 = jnp.zeros((_L,), jnp.float32)
            dw1 = jnp.zeros((_L,), jnp.float32)
            ww0 = jnp.zeros((_L,), jnp.float32)
            ww1 = jnp.zeros((_L,), jnp.float32)
            dch = []
            wch = []
            for k in range(_NCH):
                hk = hrows[i, pl.ds(k * _L, _L)]
                tk = trows[i, pl.ds(k * _L, _L)]
                wk = wrows[i, pl.ds(k * _L, _L)]
                dk = hk - tk
                if k % 2 == 0:
                    dw0 = dw0 + dk * wk
                    ww0 = ww0 + wk * wk
                else:
                    dw1 = dw1 + dk * wk
                    ww1 = ww1 + wk * wk
                dch.append(dk)
                wch.append(wk)
            c = allsum(dw0 + dw1) / jnp.maximum(allsum(ww0 + ww1), 1e-24)
            acc0 = jnp.zeros((_L,), jnp.float32)
            acc1 = jnp.zeros((_L,), jnp.float32)
            for k in range(_NCH):
                rk = rrows[i, pl.ds(k * _L, _L)]
                term = jnp.abs(dch[k] + rk - c * wch[k])
                if k % 2 == 0:
                    acc0 = acc0 + term
                else:
                    acc1 = acc1 + term
            lpos = (g % (_L // _U)) * _U + u
            svec = jnp.where(lane == lpos, -allsum(acc0 + acc1), svec)
        outv[pl.ds((g // (_L // _U)) * _L, _L)] = svec
        return svec

    gpc = _CR // _U
    svec = jnp.zeros((_L,), jnp.float32)

    def step(g, svec):
        for c in range(_NCK):
            @pl.when(g == c * gpc)
            def _(c=c):
                for cp in copies[c]:
                    cp.wait()
                if c + 2 < _NCK:
                    for cp in copies[c + 2]:
                        cp.start()
        return quad(g, svec)

    lax.fori_loop(0, _NCK * gpc, step, svec)
    pltpu.sync_copy(outv, out_hbm.at[pl.ds(base, _BPW)])


@jax.jit
def _transh_sc(h, r, t, ent_emb, rel_emb, rel_norm):
    mesh = plsc.VectorSubcoreMesh(core_axis_name="c", subcore_axis_name="s")
    run = functools.partial(
        pl.kernel,
        mesh=mesh,
        out_type=jax.ShapeDtypeStruct((_B,), jnp.float32),
        scratch_types=[
            pltpu.VMEM((_BPW,), jnp.int32),
            pltpu.VMEM((_BPW,), jnp.int32),
            pltpu.VMEM((_BPW,), jnp.int32),
            pltpu.VMEM((_BPW, _D), jnp.float32),
            pltpu.VMEM((_BPW, _D), jnp.float32),
            pltpu.VMEM((_BPW, _D), jnp.float32),
            pltpu.VMEM((_BPW, _D), jnp.float32),
            pltpu.VMEM((_BPW,), jnp.float32),
            pltpu.SemaphoreType.DMA,
            pltpu.SemaphoreType.DMA,
            pltpu.SemaphoreType.DMA,
            pltpu.SemaphoreType.DMA,
            pltpu.SemaphoreType.DMA,
        ],
    )(_tec_body)
    return run(h, r, t, ent_emb, rel_emb, rel_norm)


def kernel(h, r, t, ent_emb, rel_emb, rel_norm):
    score = _transh_sc(h.astype(jnp.int32), r.astype(jnp.int32),
                       t.astype(jnp.int32), ent_emb, rel_emb, rel_norm)
    return score.reshape(_B, 1)


# speedup vs baseline: 1.3931x; 1.0057x over previous
import functools

import jax
import jax.numpy as jnp
from jax import lax
from jax.experimental import pallas as pl
from jax.experimental.pallas import tpu as pltpu
from jax.experimental.pallas import tpu_sc as plsc

_B = 4096
_D = 128
_L = 16
_NC = 2
_NS = 16
_NW = _NC * _NS
_BPW = _B // _NW
_NCH = _D // _L


_NCK = 8
_CR = _BPW // _NCK
_U = 4


def _tec_body(h_hbm, r_hbm, t_hbm, ent_hbm, rel_hbm, nrm_hbm, out_hbm,
              hidx, ridx, tidx, hrows, trows, rrows, wrows, outv,
              sem_idx, *sems):
    wid = lax.axis_index("s") * _NC + lax.axis_index("c")
    base = wid * _BPW

    ci_h = pltpu.async_copy(h_hbm.at[pl.ds(base, _BPW)], hidx, sem_idx)
    ci_t = pltpu.async_copy(t_hbm.at[pl.ds(base, _BPW)], tidx, sem_idx)
    ci_r = pltpu.async_copy(r_hbm.at[pl.ds(base, _BPW)], ridx, sem_idx)
    ci_h.wait()
    ci_t.wait()
    ci_r.wait()

    copies = []
    for c in range(_NCK):
        sl = pl.ds(c * _CR, _CR)
        copies.append((
            pltpu.make_async_copy(ent_hbm.at[hidx.at[sl]], hrows.at[sl], sems[c]),
            pltpu.make_async_copy(ent_hbm.at[tidx.at[sl]], trows.at[sl], sems[c]),
            pltpu.make_async_copy(rel_hbm.at[ridx.at[sl]], rrows.at[sl], sems[c]),
            pltpu.make_async_copy(nrm_hbm.at[ridx.at[sl]], wrows.at[sl], sems[c]),
        ))
    for cp in copies[0]:
        cp.start()
    for cp in copies[1]:
        cp.start()

    lane = lax.broadcasted_iota(jnp.int32, (_L,), 0)
    _dnums = lax.GatherDimensionNumbers(
        offset_dims=(), collapsed_slice_dims=(0,), start_index_map=(0,))

    def shuffle(x, idx):
        return lax.gather(x, idx[:, None], _dnums, slice_sizes=(1,),
                          mode=lax.GatherScatterMode.PROMISE_IN_BOUNDS)

    def allsum(x):
        for d in (8, 4, 2, 1):
            x = x + shuffle(x, lane ^ d)
        return x

    def quad(g, svec):
        for u in range(_U):
            i = g * _U + u
            dw0 = jnp.zeros((_L,), jnp.float32)
            dw1 = jnp.zeros((_L,), jnp.float32)
            ww0 = jnp.zeros((_L,), jnp.float32)
            ww1 = jnp.zeros((_L,), jnp.float32)
            dch = []
            wch = []
            for k in range(_NCH):
                hk = hrows[i, pl.ds(k * _L, _L)]
                tk = trows[i, pl.ds(k * _L, _L)]
                wk = wrows[i, pl.ds(k * _L, _L)]
                dk = hk - tk
                if k % 2 == 0:
                    dw0 = dw0 + dk * wk
                    ww0 = ww0 + wk * wk
                else:
                    dw1 = dw1 + dk * wk
                    ww1 = ww1 + wk * wk
                dch.append(dk)
                wch.append(wk)
            c = allsum(dw0 + dw1) / jnp.maximum(allsum(ww0 + ww1), 1e-24)
            acc0 = jnp.zeros((_L,), jnp.float32)
            acc1 = jnp.zeros((_L,), jnp.float32)
            for k in range(_NCH):
                rk = rrows[i, pl.ds(k * _L, _L)]
                term = jnp.abs(dch[k] + rk - c * wch[k])
                if k % 2 == 0:
                    acc0 = acc0 - term
                else:
                    acc1 = acc1 - term
            lpos = (g % (_L // _U)) * _U + u
            svec = jnp.where(lane == lpos, allsum(acc0 + acc1), svec)
        outv[pl.ds((g // (_L // _U)) * _L, _L)] = svec
        return svec

    gpc = _CR // _U
    svec = jnp.zeros((_L,), jnp.float32)

    def step(g, svec):
        for c in range(_NCK):
            @pl.when(g == c * gpc)
            def _(c=c):
                for cp in copies[c]:
                    cp.wait()
                if c + 2 < _NCK:
                    for cp in copies[c + 2]:
                        cp.start()
        return quad(g, svec)

    lax.fori_loop(0, _NCK * gpc, step, svec)
    pltpu.sync_copy(outv, out_hbm.at[pl.ds(base, _BPW)])


@jax.jit
def _transh_sc(h, r, t, ent_emb, rel_emb, rel_norm):
    mesh = plsc.VectorSubcoreMesh(core_axis_name="c", subcore_axis_name="s")
    run = functools.partial(
        pl.kernel,
        mesh=mesh,
        out_type=jax.ShapeDtypeStruct((_B,), jnp.float32),
        scratch_types=[
            pltpu.VMEM((_BPW,), jnp.int32),
            pltpu.VMEM((_BPW,), jnp.int32),
            pltpu.VMEM((_BPW,), jnp.int32),
            pltpu.VMEM((_BPW, _D), jnp.float32),
            pltpu.VMEM((_BPW, _D), jnp.float32),
            pltpu.VMEM((_BPW, _D), jnp.float32),
            pltpu.VMEM((_BPW, _D), jnp.float32),
            pltpu.VMEM((_BPW,), jnp.float32),
        ] + [pltpu.SemaphoreType.DMA] * (1 + _NCK),
    )(_tec_body)
    return run(h, r, t, ent_emb, rel_emb, rel_norm)


def kernel(h, r, t, ent_emb, rel_emb, rel_norm):
    score = _transh_sc(h.astype(jnp.int32), r.astype(jnp.int32),
                       t.astype(jnp.int32), ent_emb, rel_emb, rel_norm)
    return score.reshape(_B, 1)
